# baseline (speedup 1.0000x reference)
import jax
import jax.numpy as jnp
from jax import lax
from jax.experimental import pallas as pl
from jax.experimental.pallas import tpu as pltpu

NUM_CLASSES = 1000
B0 = 4096
B1 = 26
BLOCK = 32
K = 8
N_STEPS = B0 // BLOCK


def _onehot_body(x_ref, o_ref, buf_ref, sem_ref):
    def outer(b, _):
        for j in range(K):
            i = b * K + j

            @pl.when(b > 0)
            def _wait():
                pltpu.make_async_copy(
                    buf_ref.at[j],
                    o_ref.at[pl.ds((i - K) * BLOCK, BLOCK)],
                    sem_ref.at[j],
                ).wait()

            idx = x_ref[pl.ds(i * BLOCK, BLOCK), :]
            iota = jax.lax.broadcasted_iota(
                jnp.int32, (BLOCK, B1, NUM_CLASSES), 2
            )
            buf_ref[j] = (iota == idx[:, :, None]).astype(jnp.int32)
            pltpu.make_async_copy(
                buf_ref.at[j],
                o_ref.at[pl.ds(i * BLOCK, BLOCK)],
                sem_ref.at[j],
            ).start()
        return 0

    lax.fori_loop(0, N_STEPS // K, outer, 0)
    for j in range(K):
        i = N_STEPS - K + j
        pltpu.make_async_copy(
            buf_ref.at[j],
            o_ref.at[pl.ds(i * BLOCK, BLOCK)],
            sem_ref.at[j],
        ).wait()


def kernel(x):
    out = pl.pallas_call(
        _onehot_body,
        in_specs=[pl.BlockSpec(memory_space=pltpu.VMEM)],
        out_specs=pl.BlockSpec(memory_space=pl.ANY),
        out_shape=jax.ShapeDtypeStruct((B0, B1, NUM_CLASSES), jnp.int32),
        scratch_shapes=[
            pltpu.VMEM((K, BLOCK, B1, NUM_CLASSES), jnp.int32),
            pltpu.SemaphoreType.DMA((K,)),
        ],
    )(x)
    return out
